# Initial kernel scaffold; baseline (speedup 1.0000x reference)
import jax
import jax.numpy as jnp
from jax.experimental import pallas as pl


def kernel(x, edge_index, edge_w, W_in, b_in, W_layers, b_layers, W_out, b_out, temp):
    raise NotImplementedError("write your pallas kernel here")


# speedup vs baseline: 1.0382x; 1.0382x over previous
import jax
import jax.numpy as jnp
from jax.experimental import pallas as pl
from jax.experimental.pallas import tpu as pltpu

N = 10000
D = 128
ROW_BLOCK = 1000


def _dense_kernel(x_ref, w_ref, b_ref, o_ref):
    o_ref[...] = (
        jax.lax.dot_general(
            x_ref[...], w_ref[...], (((1,), (1,)), ((), ())),
            preferred_element_type=jnp.float32)
        + b_ref[...]
    )


def _dense(x, W, b):
    n = x.shape[0]
    grid = (n // ROW_BLOCK,)
    return pl.pallas_call(
        _dense_kernel,
        grid=grid,
        in_specs=[
            pl.BlockSpec((ROW_BLOCK, D), lambda i: (i, 0)),
            pl.BlockSpec((D, D), lambda i: (0, 0)),
            pl.BlockSpec((1, D), lambda i: (0, 0)),
        ],
        out_specs=pl.BlockSpec((ROW_BLOCK, D), lambda i: (i, 0)),
        out_shape=jax.ShapeDtypeStruct((n, D), jnp.float32),
    )(x, W, b.reshape(1, D))


def kernel(x, edge_index, edge_w, W_in, b_in, W_layers, b_layers, W_out, b_out, temp):
    src = edge_index[0]
    dst = edge_index[1]
    h = _dense(x, W_in, b_in)
    hidden = h * temp[0]
    K = W_layers.shape[0]
    for i in range(K):
        h_lin = _dense(h, W_layers[i], b_layers[i])
        msg = h_lin[src] * edge_w[:, None]
        h = jax.ops.segment_sum(msg, dst, num_segments=N)
        h = jax.nn.relu(h)
        hidden = hidden + h * temp[i + 1]
    out = _dense(hidden, W_out, b_out)
    return out


# speedup vs baseline: 2.8571x; 2.7520x over previous
import dataclasses
import functools

import jax
import jax.numpy as jnp
from jax import lax
from jax.experimental import pallas as pl
from jax.experimental.pallas import tpu as pltpu
from jax.experimental.pallas import tpu_sc as plsc

N = 10000
D = 128
CHUNK = 128
N_TILES = 32
SC_TILES = 16
ROWS_PER_TILE = 632
N_PAD = SC_TILES * ROWS_PER_TILE
ROW_BLOCK = 1000


def _sc_body(hlin_hbm, src_hbm, dst_hbm, w_hbm, zeros_hbm, out_hbm,
             src_v, dst_v, w_v, rows_v, acc, sem, *, chunks_per_tile):
    cid = lax.axis_index("c")
    sid = lax.axis_index("s")

    row0 = sid * ROWS_PER_TILE
    pltpu.sync_copy(zeros_hbm, acc.at[pl.ds(row0, ROWS_PER_TILE)])
    plsc.subcore_barrier()

    tile_base = (cid * SC_TILES + sid) * (chunks_per_tile * CHUNK)

    @pl.loop(0, chunks_per_tile)
    def _chunk(k):
        base = tile_base + k * CHUNK
        pltpu.sync_copy(src_hbm.at[pl.ds(base, CHUNK)], src_v)
        pltpu.sync_copy(dst_hbm.at[pl.ds(base, CHUNK)], dst_v)
        pltpu.sync_copy(w_hbm.at[pl.ds(base, CHUNK)], w_v)
        pltpu.async_copy(hlin_hbm.at[src_v], rows_v, sem).wait()

        @pl.loop(0, CHUNK)
        def _edge(e):
            wsplat = plsc.load_gather(w_v, [jnp.full((16,), e, jnp.int32)])
            for c in range(D // 16):
                sl = (e, pl.ds(c * 16, 16))
                rows_v[sl] = rows_v[sl] * wsplat

        pltpu.sync_copy(rows_v, acc.at[dst_v], add=True)

    plsc.subcore_barrier()
    pltpu.sync_copy(acc.at[pl.ds(row0, ROWS_PER_TILE)],
                    out_hbm.at[cid, pl.ds(row0, ROWS_PER_TILE)])


_SC_PARAMS = pltpu.CompilerParams()
if "needs_layout_passes" in pltpu.CompilerParams.__dataclass_fields__:
    _SC_PARAMS = dataclasses.replace(_SC_PARAMS, needs_layout_passes=False)


def _sc_segment(h_lin, srcp, dstp, wp, zeros_rows, chunks_per_tile):
    kern = pl.kernel(
        functools.partial(_sc_body, chunks_per_tile=chunks_per_tile),
        out_type=jax.ShapeDtypeStruct((2, N_PAD, D), jnp.float32),
        mesh=plsc.VectorSubcoreMesh(core_axis_name="c", subcore_axis_name="s"),
        scratch_types=[
            pltpu.VMEM((CHUNK,), jnp.int32),
            pltpu.VMEM((CHUNK,), jnp.int32),
            pltpu.VMEM((CHUNK,), jnp.float32),
            pltpu.VMEM((CHUNK, D), jnp.float32),
            pltpu.VMEM_SHARED((N_PAD, D), jnp.float32),
            pltpu.SemaphoreType.DMA,
        ],
        compiler_params=_SC_PARAMS,
    )
    return kern(h_lin, srcp, dstp, wp, zeros_rows)


def _mm(a, w):
    return lax.dot_general(a, w, (((1,), (1,)), ((), ())),
                           preferred_element_type=jnp.float32)


def _in_body(x_ref, wi_ref, bi_ref, w0_ref, b0_ref, t_ref, hlin_ref, hid_ref):
    h = _mm(x_ref[...], wi_ref[...]) + bi_ref[...]
    hid_ref[...] = h * t_ref[0]
    hlin_ref[...] = _mm(h, w0_ref[...]) + b0_ref[...]


def _mid_body(p_ref, w_ref, b_ref, hin_ref, t_ref, hlin_ref, hout_ref, *, ti):
    h = jnp.maximum(p_ref[0] + p_ref[1], 0.0)
    hout_ref[...] = hin_ref[...] + h * t_ref[ti]
    hlin_ref[...] = _mm(h, w_ref[...]) + b_ref[...]


def _fin_body(p_ref, w_ref, b_ref, hin_ref, t_ref, out_ref, *, ti):
    h = jnp.maximum(p_ref[0] + p_ref[1], 0.0)
    hidden = hin_ref[...] + h * t_ref[ti]
    out_ref[...] = _mm(hidden, w_ref[...]) + b_ref[...]


_W_SPEC = pl.BlockSpec((D, D), lambda i: (0, 0))
_B_SPEC = pl.BlockSpec((1, D), lambda i: (0, 0))
_ROW_SPEC = pl.BlockSpec((ROW_BLOCK, D), lambda i: (i, 0))
_P_SPEC = pl.BlockSpec((2, ROW_BLOCK, D), lambda i: (0, i, 0))
_T_SPEC = pl.BlockSpec(memory_space=pltpu.SMEM)


def _in_call(x, W_in, b_in, W0, b0, temp):
    n = x.shape[0]
    return pl.pallas_call(
        _in_body,
        grid=(n // ROW_BLOCK,),
        in_specs=[_ROW_SPEC, _W_SPEC, _B_SPEC, _W_SPEC, _B_SPEC, _T_SPEC],
        out_specs=[_ROW_SPEC, _ROW_SPEC],
        out_shape=[jax.ShapeDtypeStruct((n, D), jnp.float32),
                   jax.ShapeDtypeStruct((n, D), jnp.float32)],
    )(x, W_in, b_in, W0, b0, temp)


def _mid_call(parts, W, b, hidden, temp, ti):
    n = hidden.shape[0]
    return pl.pallas_call(
        functools.partial(_mid_body, ti=ti),
        grid=(n // ROW_BLOCK,),
        in_specs=[_P_SPEC, _W_SPEC, _B_SPEC, _ROW_SPEC, _T_SPEC],
        out_specs=[_ROW_SPEC, _ROW_SPEC],
        out_shape=[jax.ShapeDtypeStruct((n, D), jnp.float32),
                   jax.ShapeDtypeStruct((n, D), jnp.float32)],
    )(parts, W, b, hidden, temp)


def _fin_call(parts, W, b, hidden, temp, ti):
    n = hidden.shape[0]
    return pl.pallas_call(
        functools.partial(_fin_body, ti=ti),
        grid=(n // ROW_BLOCK,),
        in_specs=[_P_SPEC, _W_SPEC, _B_SPEC, _ROW_SPEC, _T_SPEC],
        out_specs=_ROW_SPEC,
        out_shape=jax.ShapeDtypeStruct((n, D), jnp.float32),
    )(parts, W, b, hidden, temp)


def kernel(x, edge_index, edge_w, W_in, b_in, W_layers, b_layers, W_out, b_out, temp):
    K = W_layers.shape[0]
    E = edge_index.shape[1]
    chunks_per_tile = -(-E // (N_TILES * CHUNK))
    e_pad = N_TILES * chunks_per_tile * CHUNK
    pad = e_pad - E
    src = jnp.concatenate([edge_index[0], jnp.zeros((pad,), jnp.int32)])
    dst = jnp.concatenate([edge_index[1], jnp.zeros((pad,), jnp.int32)])
    wp = jnp.concatenate([edge_w, jnp.zeros((pad,), jnp.float32)])
    zeros_rows = jnp.zeros((ROWS_PER_TILE, D), jnp.float32)

    hlin, hidden = _in_call(x, W_in, b_in.reshape(1, D),
                            W_layers[0], b_layers[0].reshape(1, D), temp)
    out = None
    for i in range(K):
        parts = _sc_segment(hlin, src, dst, wp, zeros_rows, chunks_per_tile)
        if i < K - 1:
            hlin, hidden = _mid_call(parts, W_layers[i + 1],
                                     b_layers[i + 1].reshape(1, D),
                                     hidden, temp, i + 1)
        else:
            out = _fin_call(parts, W_out, b_out.reshape(1, D),
                            hidden, temp, i + 1)
    return out
